# baseline (speedup 1.0000x reference)
import functools

import jax
import jax.numpy as jnp
from jax import lax
from jax.experimental import pallas as pl
from jax.experimental.pallas import tpu as pltpu
from jax.experimental.pallas import tpu_sc as plsc

_BATCH = 2
_SEQ = 8192
_DIM = 4096
_BLK = 512
_TOPK = 64
_NBLK = _BATCH * (_SEQ // _BLK)
_NSEL = _NBLK * _TOPK


_BPG = 2


def _rank_select(scores, base):
    s_col = scores
    s_row = jnp.transpose(scores)
    gt = s_row > s_col
    eq = s_row == s_col
    jlt = (lax.broadcasted_iota(jnp.int32, (_BLK, _BLK), 1)
           < lax.broadcasted_iota(jnp.int32, (_BLK, _BLK), 0))
    rank = jnp.sum((gt | (eq & jlt)).astype(jnp.int32), axis=1, keepdims=True)
    r_row = lax.broadcasted_iota(jnp.int32, (1, _TOPK), 1)
    sel = (rank == r_row).astype(jnp.int32)
    tok = lax.broadcasted_iota(jnp.int32, (_BLK, 1), 0) + base
    return jnp.sum(sel * tok, axis=0, keepdims=True)


def _select_body(blk_off, x_ref, w_ref, b_ref, out_ref):
    g = pl.program_id(0) * _BPG + blk_off
    xb = x_ref[0]
    scores = jnp.dot(xb, w_ref[...],
                     preferred_element_type=jnp.float32) + b_ref[0, 0]
    outs = [_rank_select(scores[p * _BLK:(p + 1) * _BLK], (g + p) * _BLK)
            for p in range(_BPG)]
    out_ref[...] = jnp.concatenate(outs, axis=1)[0]


def _select_manual(xflat, w2, b2):
    nstep = _NBLK // _BPG
    rows = _BPG * _BLK
    nb = 3

    def body(x_hbm, w_ref, b_ref, out_ref, *scr):
        bufs = scr[:nb]
        sems = scr[nb:2 * nb]
        cops = [None] * nb
        for s in range(nb - 1):
            cops[s] = pltpu.make_async_copy(
                x_hbm.at[pl.ds(s * rows, rows)], bufs[s], sems[s])
            cops[s].start()
        for i in range(nstep):
            sl = i % nb
            nxt = i + nb - 1
            if nxt < nstep:
                nsl = nxt % nb
                cops[nsl] = pltpu.make_async_copy(
                    x_hbm.at[pl.ds(nxt * rows, rows)], bufs[nsl], sems[nsl])
                cops[nsl].start()
            cops[sl].wait()
            xb = bufs[sl][...]
            scores = jnp.dot(xb, w_ref[...],
                             preferred_element_type=jnp.float32) + b_ref[0, 0]
            outs = [_rank_select(scores[p * _BLK:(p + 1) * _BLK],
                                 (i * _BPG + p) * _BLK) for p in range(_BPG)]
            out_ref[pl.ds(i * _BPG * _TOPK, _BPG * _TOPK)] = (
                jnp.concatenate(outs, axis=1)[0])

    return pl.pallas_call(
        body,
        in_specs=[
            pl.BlockSpec(memory_space=pl.ANY),
            pl.BlockSpec(memory_space=pltpu.VMEM),
            pl.BlockSpec(memory_space=pltpu.SMEM),
        ],
        out_specs=pl.BlockSpec(memory_space=pltpu.VMEM),
        out_shape=jax.ShapeDtypeStruct((_NBLK * _TOPK,), jnp.int32),
        scratch_shapes=(
            [pltpu.VMEM((rows, _DIM), jnp.float32) for _ in range(nb)]
            + [pltpu.SemaphoreType.DMA for _ in range(nb)]
        ),
        compiler_params=pltpu.CompilerParams(
            vmem_limit_bytes=100 * 1024 * 1024),
    )(xflat, w2, b2)


def _select_indices(xg, w2, b2, blk_off, nblk):
    ng = nblk // _BPG
    idx = pl.pallas_call(
        functools.partial(_select_body, blk_off),
        grid=(ng,),
        in_specs=[
            pl.BlockSpec((1, _BPG * _BLK, _DIM),
                         lambda i: (i + blk_off // _BPG, 0, 0)),
            pl.BlockSpec((_DIM, 1), lambda i: (0, 0)),
            pl.BlockSpec(memory_space=pltpu.SMEM),
        ],
        out_specs=pl.BlockSpec((_BPG * _TOPK,), lambda i: (i,)),
        out_shape=jax.ShapeDtypeStruct((nblk * _TOPK,), jnp.int32),
        compiler_params=pltpu.CompilerParams(
            vmem_limit_bytes=100 * 1024 * 1024),
    )(xg, w2, b2)
    return idx


def _gather(xflat, idx, nsel):
    rows_per_sub = nsel // 32
    chunk = 8
    nchunks = rows_per_sub // chunk

    nbuf = 3

    @functools.partial(
        pl.kernel,
        mesh=plsc.VectorSubcoreMesh(core_axis_name="c", subcore_axis_name="s"),
        out_type=jax.ShapeDtypeStruct((nsel, _DIM), jnp.float32),
        scratch_types=(
            [pltpu.VMEM((chunk,), jnp.int32) for _ in range(nbuf)]
            + [pltpu.VMEM((chunk, _DIM), jnp.float32) for _ in range(nbuf)]
            + [pltpu.SemaphoreType.DMA for _ in range(2 * nbuf)]
        ),
    )
    def gk(x_hbm, idx_hbm, out_hbm, *scr):
        idx_v = scr[:nbuf]
        rows_v = scr[nbuf:2 * nbuf]
        gsem = scr[2 * nbuf:3 * nbuf]
        wsem = scr[3 * nbuf:4 * nbuf]
        wid = lax.axis_index("s") * 2 + lax.axis_index("c")
        base = wid * rows_per_sub
        gops = [None] * nbuf
        wops = [None] * nbuf
        for c in range(nbuf - 1):
            pltpu.sync_copy(idx_hbm.at[pl.ds(base + c * chunk, chunk)], idx_v[c])
            gops[c] = pltpu.async_copy(x_hbm.at[idx_v[c]], rows_v[c], gsem[c])
        for c in range(nchunks):
            b = c % nbuf
            nxt = c + nbuf - 1
            nb = nxt % nbuf
            if nxt < nchunks:
                if wops[nb] is not None:
                    wops[nb].wait()
                off = base + nxt * chunk
                pltpu.sync_copy(idx_hbm.at[pl.ds(off, chunk)], idx_v[nb])
                gops[nb] = pltpu.async_copy(x_hbm.at[idx_v[nb]], rows_v[nb], gsem[nb])
            gops[b].wait()
            wops[b] = pltpu.async_copy(
                rows_v[b], out_hbm.at[pl.ds(base + c * chunk, chunk)], wsem[b])
        for c in range(max(0, nchunks - nbuf), nchunks):
            wops[c % nbuf].wait()

    return gk(xflat, idx)


def kernel(x, W, b):
    xg = x.reshape(_NBLK // _BPG, _BPG * _BLK, _DIM)
    xflat = x.reshape(_BATCH * _SEQ, _DIM)
    b2 = b.reshape(1, 1)
    idx = _select_manual(xflat, W.reshape(_DIM, 1), b2)
    out = _gather(xflat, idx, _NSEL)
    return out.reshape(_BATCH, _NSEL // _BATCH, _DIM)


# speedup vs baseline: 1.0811x; 1.0811x over previous
import functools

import jax
import jax.numpy as jnp
from jax import lax
from jax.experimental import pallas as pl
from jax.experimental.pallas import tpu as pltpu
from jax.experimental.pallas import tpu_sc as plsc

_BATCH = 2
_SEQ = 8192
_DIM = 4096
_BLK = 512
_TOPK = 64
_NBLK = _BATCH * (_SEQ // _BLK)
_NSEL = _NBLK * _TOPK


_BPG = 2


def _rank_select(scores, base):
    s_col = scores
    s_row = jnp.transpose(scores)
    gt = s_row > s_col
    eq = s_row == s_col
    jlt = (lax.broadcasted_iota(jnp.int32, (_BLK, _BLK), 1)
           < lax.broadcasted_iota(jnp.int32, (_BLK, _BLK), 0))
    rank = jnp.sum((gt | (eq & jlt)).astype(jnp.int32), axis=1, keepdims=True)
    r_row = lax.broadcasted_iota(jnp.int32, (1, _TOPK), 1)
    sel = (rank == r_row).astype(jnp.int32)
    tok = lax.broadcasted_iota(jnp.int32, (_BLK, 1), 0) + base
    return jnp.sum(sel * tok, axis=0, keepdims=True)


def _select_body(blk_off, x_ref, w_ref, b_ref, out_ref):
    g = pl.program_id(0) * _BPG + blk_off
    xb = x_ref[0]
    scores = jnp.dot(xb, w_ref[...],
                     preferred_element_type=jnp.float32) + b_ref[0, 0]
    outs = [_rank_select(scores[p * _BLK:(p + 1) * _BLK], (g + p) * _BLK)
            for p in range(_BPG)]
    out_ref[...] = jnp.concatenate(outs, axis=1)[0]


def _select_manual(xflat, w2, b2):
    nstep = _NBLK // _BPG
    rows = _BPG * _BLK
    nb = 3

    def body(x_hbm, w_ref, b_ref, out_ref, *scr):
        bufs = scr[:nb]
        sems = scr[nb:2 * nb]
        cops = [None] * nb
        for s in range(nb - 1):
            cops[s] = pltpu.make_async_copy(
                x_hbm.at[pl.ds(s * rows, rows)], bufs[s], sems[s])
            cops[s].start()
        for i in range(nstep):
            sl = i % nb
            nxt = i + nb - 1
            if nxt < nstep:
                nsl = nxt % nb
                cops[nsl] = pltpu.make_async_copy(
                    x_hbm.at[pl.ds(nxt * rows, rows)], bufs[nsl], sems[nsl])
                cops[nsl].start()
            cops[sl].wait()
            xb = bufs[sl][...]
            scores = jnp.dot(xb, w_ref[...],
                             preferred_element_type=jnp.float32) + b_ref[0, 0]
            outs = [_rank_select(scores[p * _BLK:(p + 1) * _BLK],
                                 (i * _BPG + p) * _BLK) for p in range(_BPG)]
            out_ref[pl.ds(i * _BPG * _TOPK, _BPG * _TOPK)] = (
                jnp.concatenate(outs, axis=1)[0])

    return pl.pallas_call(
        body,
        in_specs=[
            pl.BlockSpec(memory_space=pl.ANY),
            pl.BlockSpec(memory_space=pltpu.VMEM),
            pl.BlockSpec(memory_space=pltpu.SMEM),
        ],
        out_specs=pl.BlockSpec(memory_space=pltpu.VMEM),
        out_shape=jax.ShapeDtypeStruct((_NBLK * _TOPK,), jnp.int32),
        scratch_shapes=(
            [pltpu.VMEM((rows, _DIM), jnp.float32) for _ in range(nb)]
            + [pltpu.SemaphoreType.DMA for _ in range(nb)]
        ),
        compiler_params=pltpu.CompilerParams(
            vmem_limit_bytes=100 * 1024 * 1024),
    )(xflat, w2, b2)


def _select_indices(xg, w2, b2, blk_off, nblk):
    ng = nblk // _BPG
    idx = pl.pallas_call(
        functools.partial(_select_body, blk_off),
        grid=(ng,),
        in_specs=[
            pl.BlockSpec((1, _BPG * _BLK, _DIM),
                         lambda i: (i + blk_off // _BPG, 0, 0)),
            pl.BlockSpec((_DIM, 1), lambda i: (0, 0)),
            pl.BlockSpec(memory_space=pltpu.SMEM),
        ],
        out_specs=pl.BlockSpec((_BPG * _TOPK,), lambda i: (i,)),
        out_shape=jax.ShapeDtypeStruct((nblk * _TOPK,), jnp.int32),
        compiler_params=pltpu.CompilerParams(
            vmem_limit_bytes=100 * 1024 * 1024),
    )(xg, w2, b2)
    return idx


def _gather(xflat, idx, nsel):
    rows_per_sub = nsel // 32
    chunk = 8
    nchunks = rows_per_sub // chunk

    nbuf = 3

    @functools.partial(
        pl.kernel,
        mesh=plsc.VectorSubcoreMesh(core_axis_name="c", subcore_axis_name="s"),
        out_type=jax.ShapeDtypeStruct((nsel, _DIM), jnp.float32),
        scratch_types=(
            [pltpu.VMEM((chunk,), jnp.int32) for _ in range(nbuf)]
            + [pltpu.VMEM((chunk, _DIM), jnp.float32) for _ in range(nbuf)]
            + [pltpu.SemaphoreType.DMA for _ in range(2 * nbuf)]
        ),
    )
    def gk(x_hbm, idx_hbm, out_hbm, *scr):
        idx_v = scr[:nbuf]
        rows_v = scr[nbuf:2 * nbuf]
        gsem = scr[2 * nbuf:3 * nbuf]
        wsem = scr[3 * nbuf:4 * nbuf]
        wid = lax.axis_index("s") * 2 + lax.axis_index("c")
        base = wid * rows_per_sub
        gops = [None] * nbuf
        wops = [None] * nbuf
        for c in range(nbuf - 1):
            pltpu.sync_copy(idx_hbm.at[pl.ds(base + c * chunk, chunk)], idx_v[c])
            gops[c] = pltpu.async_copy(x_hbm.at[idx_v[c]], rows_v[c], gsem[c])
        for c in range(nchunks):
            b = c % nbuf
            nxt = c + nbuf - 1
            nb = nxt % nbuf
            if nxt < nchunks:
                if wops[nb] is not None:
                    wops[nb].wait()
                off = base + nxt * chunk
                pltpu.sync_copy(idx_hbm.at[pl.ds(off, chunk)], idx_v[nb])
                gops[nb] = pltpu.async_copy(x_hbm.at[idx_v[nb]], rows_v[nb], gsem[nb])
            gops[b].wait()
            wops[b] = pltpu.async_copy(
                rows_v[b], out_hbm.at[pl.ds(base + c * chunk, chunk)], wsem[b])
        for c in range(max(0, nchunks - nbuf), nchunks):
            wops[c % nbuf].wait()

    return gk(xflat, idx)


def kernel(x, W, b):
    xg = x.reshape(_NBLK // _BPG, _BPG * _BLK, _DIM)
    xflat = x.reshape(_BATCH * _SEQ, _DIM)
    b2 = b.reshape(1, 1)
    idx = _select_indices(xg, W.reshape(_DIM, 1), b2, 0, _NBLK)
    out = _gather(xflat, idx, _NSEL)
    return out.reshape(_BATCH, _NSEL // _BATCH, _DIM)
